# Initial kernel scaffold; baseline (speedup 1.0000x reference)
import jax
import jax.numpy as jnp
from jax.experimental import pallas as pl


def kernel(words, contexts, word_table, context_table):
    raise NotImplementedError("write your pallas kernel here")


# speedup vs baseline: 1.1120x; 1.1120x over previous
import functools

import jax
import jax.numpy as jnp
from jax import lax
from jax.experimental import pallas as pl
from jax.experimental.pallas import tpu as pltpu
from jax.experimental.pallas import tpu_sc as plsc

VOCAB = 100000
EMBED = 128
BATCH = 4096
SEQ = 20
TOTAL = BATCH * SEQ
CHUNK = 128
NUM_CORES = 2
NUM_SUBCORES = 16
NW = NUM_CORES * NUM_SUBCORES
ROWS = TOTAL // CHUNK
ROWS_PER_W = ROWS // NW


def _make_kernel():
    mesh = plsc.VectorSubcoreMesh(core_axis_name="c", subcore_axis_name="s")

    @functools.partial(
        pl.kernel,
        mesh=mesh,
        out_type=(
            jax.ShapeDtypeStruct((TOTAL, EMBED), jnp.float32),
            jax.ShapeDtypeStruct((TOTAL, EMBED), jnp.float32),
        ),
        scratch_types=[
            pltpu.VMEM((ROWS_PER_W * CHUNK,), jnp.int32),
            pltpu.VMEM((ROWS_PER_W * CHUNK,), jnp.int32),
            pltpu.VMEM((CHUNK, EMBED), jnp.float32),
            pltpu.VMEM((CHUNK, EMBED), jnp.float32),
            pltpu.SemaphoreType.DMA,
            pltpu.SemaphoreType.DMA,
        ],
    )
    def k(wtab, ctab, widx, cidx, wout, cout,
          idxw_v, idxc_v, bufw, bufc, gsem, ssem):
        wid = lax.axis_index("s") * NUM_CORES + lax.axis_index("c")
        base = pl.multiple_of(wid * (ROWS_PER_W * CHUNK), ROWS_PER_W * CHUNK)

        pltpu.sync_copy(widx.at[pl.ds(base, ROWS_PER_W * CHUNK)], idxw_v)
        pltpu.sync_copy(cidx.at[pl.ds(base, ROWS_PER_W * CHUNK)], idxc_v)

        def body(j, carry):
            off = pl.multiple_of(j * CHUNK, CHUNK)
            out_base = pl.multiple_of(base + j * CHUNK, CHUNK)
            iw = idxw_v.at[pl.ds(off, CHUNK)]
            ic = idxc_v.at[pl.ds(off, CHUNK)]
            pltpu.async_copy(wtab.at[iw], bufw, gsem).wait()
            pltpu.sync_copy(bufw, wout.at[pl.ds(out_base, CHUNK)])
            pltpu.async_copy(ctab.at[ic], bufc, ssem).wait()
            pltpu.sync_copy(bufc, cout.at[pl.ds(out_base, CHUNK)])
            return carry

        lax.fori_loop(0, ROWS_PER_W, body, 0)

    return k


_sc_gather = _make_kernel()


@jax.jit
def kernel(words, contexts, word_table, context_table):
    widx = words.astype(jnp.int32).reshape(TOTAL)
    cidx = contexts.astype(jnp.int32).reshape(TOTAL)
    w_flat, c_flat = _sc_gather(word_table, context_table, widx, cidx)
    return (w_flat.reshape(BATCH, SEQ, EMBED), c_flat.reshape(BATCH, SEQ, EMBED))


# speedup vs baseline: 1.2577x; 1.1310x over previous
import functools

import jax
import jax.numpy as jnp
from jax import lax
from jax.experimental import pallas as pl
from jax.experimental.pallas import tpu as pltpu
from jax.experimental.pallas import tpu_sc as plsc

VOCAB = 100000
EMBED = 128
BATCH = 4096
SEQ = 20
TOTAL = BATCH * SEQ
CHUNK = 128
NUM_CORES = 2
NUM_SUBCORES = 16
NW = NUM_CORES * NUM_SUBCORES
ROWS = TOTAL // CHUNK
ROWS_PER_W = ROWS // NW


def _make_kernel():
    mesh = plsc.VectorSubcoreMesh(core_axis_name="c", subcore_axis_name="s")

    @functools.partial(
        pl.kernel,
        mesh=mesh,
        out_type=(
            jax.ShapeDtypeStruct((TOTAL, EMBED), jnp.float32),
            jax.ShapeDtypeStruct((TOTAL, EMBED), jnp.float32),
        ),
        scratch_types=[
            pltpu.VMEM((ROWS_PER_W * CHUNK,), jnp.int32),
            pltpu.VMEM((ROWS_PER_W * CHUNK,), jnp.int32),
            pltpu.VMEM((CHUNK, EMBED), jnp.float32),
            pltpu.VMEM((CHUNK, EMBED), jnp.float32),
            pltpu.VMEM((CHUNK, EMBED), jnp.float32),
            pltpu.VMEM((CHUNK, EMBED), jnp.float32),
            pltpu.SemaphoreType.DMA,
            pltpu.SemaphoreType.DMA,
            pltpu.SemaphoreType.DMA,
            pltpu.SemaphoreType.DMA,
        ],
    )
    def k(wtab, ctab, widx, cidx, wout, cout,
          idxw_v, idxc_v, bw0, bw1, bc0, bc1, gw, gc, sw, sc):
        wid = lax.axis_index("s") * NUM_CORES + lax.axis_index("c")
        base = pl.multiple_of(wid * (ROWS_PER_W * CHUNK), ROWS_PER_W * CHUNK)
        bufw = (bw0, bw1)
        bufc = (bc0, bc1)

        pltpu.sync_copy(widx.at[pl.ds(base, ROWS_PER_W * CHUNK)], idxw_v)
        pltpu.sync_copy(cidx.at[pl.ds(base, ROWS_PER_W * CHUNK)], idxc_v)

        def iw(j):
            return idxw_v.at[pl.ds(pl.multiple_of(j * CHUNK, CHUNK), CHUNK)]

        def ic(j):
            return idxc_v.at[pl.ds(pl.multiple_of(j * CHUNK, CHUNK), CHUNK)]

        def wslab(j):
            return wout.at[pl.ds(pl.multiple_of(base + j * CHUNK, CHUNK), CHUNK)]

        def cslab(j):
            return cout.at[pl.ds(pl.multiple_of(base + j * CHUNK, CHUNK), CHUNK)]

        pltpu.async_copy(wtab.at[iw(0)], bufw[0], gw)
        pltpu.async_copy(ctab.at[ic(0)], bufc[0], gc)
        pltpu.async_copy(wtab.at[iw(1)], bufw[1], gw)
        pltpu.async_copy(ctab.at[ic(1)], bufc[1], gc)

        pltpu.make_async_copy(wtab.at[iw(0)], bufw[0], gw).wait()
        pltpu.async_copy(bufw[0], wslab(0), sw)
        pltpu.make_async_copy(ctab.at[ic(0)], bufc[0], gc).wait()
        pltpu.async_copy(bufc[0], cslab(0), sc)

        def body(jj, carry):
            for d in range(2):
                j = 2 * jj + 1 + d
                b = bufw[(1 + d) % 2]
                cb = bufc[(1 + d) % 2]
                ob = bufw[d % 2]
                ocb = bufc[d % 2]
                pltpu.make_async_copy(ob, wslab(j - 1), sw).wait()
                pltpu.async_copy(wtab.at[iw(j + 1)], ob, gw)
                pltpu.make_async_copy(ocb, cslab(j - 1), sc).wait()
                pltpu.async_copy(ctab.at[ic(j + 1)], ocb, gc)
                pltpu.make_async_copy(wtab.at[iw(j)], b, gw).wait()
                pltpu.async_copy(b, wslab(j), sw)
                pltpu.make_async_copy(ctab.at[ic(j)], cb, gc).wait()
                pltpu.async_copy(cb, cslab(j), sc)
            return carry

        lax.fori_loop(0, (ROWS_PER_W - 2) // 2, body, 0)

        last = ROWS_PER_W - 1
        pltpu.make_async_copy(bufw[0], wslab(last - 1), sw).wait()
        pltpu.make_async_copy(bufc[0], cslab(last - 1), sc).wait()
        pltpu.make_async_copy(wtab.at[iw(last)], bufw[1], gw).wait()
        pltpu.async_copy(bufw[1], wslab(last), sw)
        pltpu.make_async_copy(ctab.at[ic(last)], bufc[1], gc).wait()
        pltpu.async_copy(bufc[1], cslab(last), sc)
        pltpu.make_async_copy(bufw[1], wslab(last), sw).wait()
        pltpu.make_async_copy(bufc[1], cslab(last), sc).wait()

    return k


_sc_gather = _make_kernel()


@jax.jit
def kernel(words, contexts, word_table, context_table):
    widx = words.astype(jnp.int32).reshape(TOTAL)
    cidx = contexts.astype(jnp.int32).reshape(TOTAL)
    w_flat, c_flat = _sc_gather(word_table, context_table, widx, cidx)
    return (w_flat.reshape(BATCH, SEQ, EMBED), c_flat.reshape(BATCH, SEQ, EMBED))


# speedup vs baseline: 1.9144x; 1.5222x over previous
import functools

import jax
import jax.numpy as jnp
from jax import lax
from jax.experimental import pallas as pl
from jax.experimental.pallas import tpu as pltpu
from jax.experimental.pallas import tpu_sc as plsc

VOCAB = 100000
EMBED = 128
BATCH = 4096
SEQ = 20
NUM_CORES = 2
NUM_SUBCORES = 16
NW = NUM_CORES * NUM_SUBCORES
B_PER_W = BATCH // NW
CB = 4
NCHUNK = B_PER_W // CB


def _make_kernel():
    mesh = plsc.VectorSubcoreMesh(core_axis_name="c", subcore_axis_name="s")

    @functools.partial(
        pl.kernel,
        mesh=mesh,
        out_type=(
            jax.ShapeDtypeStruct((BATCH, SEQ, EMBED), jnp.float32),
            jax.ShapeDtypeStruct((BATCH, SEQ, EMBED), jnp.float32),
        ),
        scratch_types=[
            pltpu.VMEM((B_PER_W * SEQ,), jnp.int32),
            pltpu.VMEM((B_PER_W * SEQ,), jnp.int32),
            pltpu.VMEM((CB * SEQ, EMBED), jnp.float32),
            pltpu.VMEM((CB * SEQ, EMBED), jnp.float32),
            pltpu.VMEM((CB * SEQ, EMBED), jnp.float32),
            pltpu.VMEM((CB * SEQ, EMBED), jnp.float32),
            pltpu.SemaphoreType.DMA,
            pltpu.SemaphoreType.DMA,
            pltpu.SemaphoreType.DMA,
            pltpu.SemaphoreType.DMA,
        ],
    )
    def k(wtab, ctab, widx, cidx, wout, cout,
          idxw_v, idxc_v, bw0, bw1, bc0, bc1, gw, gc, sw, sc):
        wid = lax.axis_index("s") * NUM_CORES + lax.axis_index("c")
        b0 = pl.multiple_of(wid * B_PER_W, B_PER_W)
        bufw = (bw0, bw1)
        bufc = (bc0, bc1)

        i0 = pl.multiple_of(wid * (B_PER_W * SEQ), B_PER_W * SEQ)
        pltpu.sync_copy(widx.at[pl.ds(i0, B_PER_W * SEQ)], idxw_v)
        pltpu.sync_copy(cidx.at[pl.ds(i0, B_PER_W * SEQ)], idxc_v)

        def iw(j):
            return idxw_v.at[pl.ds(pl.multiple_of(j * CB * SEQ, CB * SEQ), CB * SEQ)]

        def ic(j):
            return idxc_v.at[pl.ds(pl.multiple_of(j * CB * SEQ, CB * SEQ), CB * SEQ)]

        def store(buf, out, j):
            for kk in range(CB):
                pltpu.async_copy(buf.at[pl.ds(kk * SEQ, SEQ)],
                                 out.at[b0 + j * CB + kk], sem_of[id(buf)])

        def drain(buf, out, j):
            for kk in range(CB):
                pltpu.make_async_copy(buf.at[pl.ds(kk * SEQ, SEQ)],
                                      out.at[b0 + j * CB + kk],
                                      sem_of[id(buf)]).wait()

        sem_of = {id(bw0): sw, id(bw1): sw, id(bc0): sc, id(bc1): sc}

        pltpu.async_copy(wtab.at[iw(0)], bufw[0], gw)
        pltpu.async_copy(ctab.at[ic(0)], bufc[0], gc)
        pltpu.async_copy(wtab.at[iw(1)], bufw[1], gw)
        pltpu.async_copy(ctab.at[ic(1)], bufc[1], gc)

        pltpu.make_async_copy(wtab.at[iw(0)], bufw[0], gw).wait()
        store(bufw[0], wout, 0)
        pltpu.make_async_copy(ctab.at[ic(0)], bufc[0], gc).wait()
        store(bufc[0], cout, 0)

        def body(jj, carry):
            for d in range(2):
                j = 2 * jj + 1 + d
                b = bufw[(1 + d) % 2]
                cbuf = bufc[(1 + d) % 2]
                ob = bufw[d % 2]
                ocb = bufc[d % 2]
                drain(ob, wout, j - 1)
                pltpu.async_copy(wtab.at[iw(j + 1)], ob, gw)
                drain(ocb, cout, j - 1)
                pltpu.async_copy(ctab.at[ic(j + 1)], ocb, gc)
                pltpu.make_async_copy(wtab.at[iw(j)], b, gw).wait()
                store(b, wout, j)
                pltpu.make_async_copy(ctab.at[ic(j)], cbuf, gc).wait()
                store(cbuf, cout, j)
            return carry

        lax.fori_loop(0, (NCHUNK - 2) // 2, body, 0)

        last = NCHUNK - 1
        drain(bufw[0], wout, last - 1)
        drain(bufc[0], cout, last - 1)
        pltpu.make_async_copy(wtab.at[iw(last)], bufw[1], gw).wait()
        store(bufw[1], wout, last)
        pltpu.make_async_copy(ctab.at[ic(last)], bufc[1], gc).wait()
        store(bufc[1], cout, last)
        drain(bufw[1], wout, last)
        drain(bufc[1], cout, last)

    return k


_sc_gather = _make_kernel()


@jax.jit
def kernel(words, contexts, word_table, context_table):
    widx = words.astype(jnp.int32).reshape(BATCH * SEQ)
    cidx = contexts.astype(jnp.int32).reshape(BATCH * SEQ)
    return _sc_gather(word_table, context_table, widx, cidx)


# speedup vs baseline: 3.6220x; 1.8919x over previous
import functools

import jax
import jax.numpy as jnp
from jax import lax
from jax.experimental import pallas as pl
from jax.experimental.pallas import tpu as pltpu
from jax.experimental.pallas import tpu_sc as plsc

VOCAB = 100000
EMBED = 128
BATCH = 4096
SEQ = 20
NUM_CORES = 2
NUM_SUBCORES = 16
NW = NUM_CORES * NUM_SUBCORES
B_PER_W = BATCH // NW
SLAB = SEQ * B_PER_W


def _make_kernel():
    mesh = plsc.VectorSubcoreMesh(core_axis_name="c", subcore_axis_name="s")

    @functools.partial(
        pl.kernel,
        mesh=mesh,
        out_type=(
            jax.ShapeDtypeStruct((SEQ, BATCH, EMBED), jnp.float32),
            jax.ShapeDtypeStruct((SEQ, BATCH, EMBED), jnp.float32),
        ),
        scratch_types=[
            pltpu.VMEM((SLAB,), jnp.int32),
            pltpu.VMEM((SLAB,), jnp.int32),
            pltpu.VMEM((B_PER_W, EMBED), jnp.float32),
            pltpu.VMEM((B_PER_W, EMBED), jnp.float32),
            pltpu.VMEM((B_PER_W, EMBED), jnp.float32),
            pltpu.VMEM((B_PER_W, EMBED), jnp.float32),
            pltpu.SemaphoreType.DMA,
            pltpu.SemaphoreType.DMA,
            pltpu.SemaphoreType.DMA,
            pltpu.SemaphoreType.DMA,
        ],
    )
    def k(wtab, ctab, widx, cidx, wout, cout,
          idxw_v, idxc_v, bw0, bw1, bc0, bc1, gw, gc, sw, sc):
        wid = lax.axis_index("s") * NUM_CORES + lax.axis_index("c")
        b0 = pl.multiple_of(wid * B_PER_W, B_PER_W)
        bufw = (bw0, bw1)
        bufc = (bc0, bc1)
        sem_of = {id(bw0): sw, id(bw1): sw, id(bc0): sc, id(bc1): sc}

        i0 = pl.multiple_of(wid * SLAB, SLAB)
        pltpu.sync_copy(widx.at[pl.ds(i0, SLAB)], idxw_v)
        pltpu.sync_copy(cidx.at[pl.ds(i0, SLAB)], idxc_v)

        def iw(j):
            return idxw_v.at[pl.ds(pl.multiple_of(j * B_PER_W, B_PER_W), B_PER_W)]

        def ic(j):
            return idxc_v.at[pl.ds(pl.multiple_of(j * B_PER_W, B_PER_W), B_PER_W)]

        def store(buf, out, j):
            pltpu.async_copy(buf, out.at[j, pl.ds(b0, B_PER_W)], sem_of[id(buf)])

        def drain(buf, out, j):
            pltpu.make_async_copy(buf, out.at[j, pl.ds(b0, B_PER_W)],
                                  sem_of[id(buf)]).wait()

        pltpu.async_copy(wtab.at[iw(0)], bufw[0], gw)
        pltpu.async_copy(ctab.at[ic(0)], bufc[0], gc)
        pltpu.async_copy(wtab.at[iw(1)], bufw[1], gw)
        pltpu.async_copy(ctab.at[ic(1)], bufc[1], gc)

        pltpu.make_async_copy(wtab.at[iw(0)], bufw[0], gw).wait()
        store(bufw[0], wout, 0)
        pltpu.make_async_copy(ctab.at[ic(0)], bufc[0], gc).wait()
        store(bufc[0], cout, 0)

        def body(jj, carry):
            for d in range(2):
                j = 2 * jj + 1 + d
                b = bufw[(1 + d) % 2]
                cbuf = bufc[(1 + d) % 2]
                ob = bufw[d % 2]
                ocb = bufc[d % 2]
                drain(ob, wout, j - 1)
                pltpu.async_copy(wtab.at[iw(j + 1)], ob, gw)
                drain(ocb, cout, j - 1)
                pltpu.async_copy(ctab.at[ic(j + 1)], ocb, gc)
                pltpu.make_async_copy(wtab.at[iw(j)], b, gw).wait()
                store(b, wout, j)
                pltpu.make_async_copy(ctab.at[ic(j)], cbuf, gc).wait()
                store(cbuf, cout, j)
            return carry

        lax.fori_loop(0, (SEQ - 2) // 2, body, 0)

        last = SEQ - 1
        drain(bufw[0], wout, last - 1)
        drain(bufc[0], cout, last - 1)
        pltpu.make_async_copy(wtab.at[iw(last)], bufw[1], gw).wait()
        store(bufw[1], wout, last)
        pltpu.make_async_copy(ctab.at[ic(last)], bufc[1], gc).wait()
        store(bufc[1], cout, last)
        drain(bufw[1], wout, last)
        drain(bufc[1], cout, last)

    return k


_sc_gather = _make_kernel()


def _rearrange(idx):
    return (idx.astype(jnp.int32)
            .reshape(NW, B_PER_W, SEQ)
            .transpose(0, 2, 1)
            .reshape(NW * SLAB))


@jax.jit
def kernel(words, contexts, word_table, context_table):
    widx = _rearrange(words)
    cidx = _rearrange(contexts)
    w_t, c_t = _sc_gather(word_table, context_table, widx, cidx)
    return (w_t.transpose(1, 0, 2), c_t.transpose(1, 0, 2))
